# baseline (speedup 1.0000x reference)
import functools

import jax
import jax.numpy as jnp
from jax import lax
from jax.experimental import pallas as pl
from jax.experimental.pallas import tpu as pltpu
from jax.experimental.pallas import tpu_sc as plsc

VOCAB = 1000000
EMBED_DIM = 64
BATCH = 16384
CTX = 20

NC = 2
NS = 16
LANES = 16
NW = NC * NS
BPW = BATCH // NW
CB = 32
NCH = BPW // CB
IPC = CB * CTX
IDX_W = 128
NSTR = IPC // IDX_W
VPD = EMBED_DIM // LANES


def _cbow_body(idx_hbm, table_hbm, out_hbm, idx_v, rows_v, out_v, sem):
    wid = lax.axis_index("s") * NC + lax.axis_index("c")
    inv_ctx = jnp.float32(1.0 / CTX)
    rows_per_w = BPW * CTX // IDX_W

    pltpu.sync_copy(idx_hbm.at[pl.ds(wid * rows_per_w, rows_per_w)], idx_v)

    def chunk_body(c, _):
        copies = [
            pltpu.async_copy(
                table_hbm.at[idx_v.at[c * NSTR + k]],
                rows_v.at[pl.ds(k * IDX_W, IDX_W)],
                sem,
            )
            for k in range(NSTR)
        ]
        for cp in copies:
            cp.wait()

        def item_body(i, _):
            base = i * CTX
            for q in range(VPD):
                sl = pl.ds(q * LANES, LANES)
                acc = rows_v[base, sl]
                for j in range(1, CTX):
                    acc = acc + rows_v[base + j, sl]
                out_v[i, sl] = acc * inv_ctx
            return 0

        lax.fori_loop(0, CB, item_body, 0)

        out_row0 = wid * BPW + c * CB
        pltpu.sync_copy(out_v, out_hbm.at[pl.ds(out_row0, CB)])
        return 0

    lax.fori_loop(0, NCH, chunk_body, 0)


@jax.jit
def _cbow(idx2d, embeddings):
    mesh = plsc.VectorSubcoreMesh(
        core_axis_name="c", subcore_axis_name="s", num_cores=NC, num_subcores=NS
    )
    return pl.kernel(
        _cbow_body,
        out_type=jax.ShapeDtypeStruct((BATCH, EMBED_DIM), jnp.float32),
        mesh=mesh,
        scratch_types=[
            pltpu.VMEM((BPW * CTX // IDX_W, IDX_W), jnp.int32),
            pltpu.VMEM((IPC, EMBED_DIM), jnp.float32),
            pltpu.VMEM((CB, EMBED_DIM), jnp.float32),
            pltpu.SemaphoreType.DMA,
        ],
        compiler_params=pltpu.CompilerParams(use_tc_tiling_on_sc=False),
        name="cbow_sc",
    )(idx2d, embeddings)


def kernel(context_idxs, embeddings):
    idx2d = context_idxs.astype(jnp.int32).reshape(BATCH * CTX // IDX_W, IDX_W)
    return _cbow(idx2d, embeddings)
